# baseline (speedup 1.0000x reference)
import jax
import jax.numpy as jnp
from jax import lax
from jax.experimental import pallas as pl
from jax.experimental.pallas import tpu as pltpu
from jax.experimental.pallas import tpu_sc as plsc

B, S = 4, 8192
TOK = B * S
IN_D = 768
PROJ_D = 512
EMB_D = 128
OUT_D = PROJ_D + 2 * EMB_D

NC, NS = 2, 16
NW = NC * NS
TPW = TOK // NW
CH = 128
NCH = TPW // CH
NBUF = 6

M_TILE = 2048


def _sc_scatter_body(tag_idx_hbm, tag_tab_hbm, out_hbm,
                     idx_t, r0, r1, r2, r3, r4, r5,
                     s0, s1, s2, s3, s4, s5,
                     t0, t1, t2, t3, t4, t5):
    wid = lax.axis_index("s") * NC + lax.axis_index("c")
    base = wid * NCH
    pltpu.sync_copy(tag_idx_hbm.at[pl.ds(base, NCH)], idx_t)
    rows = (r0, r1, r2, r3, r4, r5)
    gsems = (s0, s1, s2, s3, s4, s5)
    ssems = (t0, t1, t2, t3, t4, t5)

    def fire(j):
        return pltpu.async_copy(
            tag_tab_hbm.at[idx_t.at[j]], rows[j % NBUF], gsems[j % NBUF])

    cps = {}
    sts = {}
    for j in range(NBUF):
        cps[j] = fire(j)
    for j in range(NCH):
        cps.pop(j).wait()
        sts[j] = pltpu.async_copy(
            rows[j % NBUF],
            out_hbm.at[pl.ds((base + j) * CH, CH), pl.ds(PROJ_D, EMB_D)],
            ssems[j % NBUF])
        if j + NBUF < NCH:
            sts.pop(j).wait()
            cps[j + NBUF] = fire(j + NBUF)
    for j in sorted(sts):
        sts[j].wait()


def _sc_scatter_tags(tag_idx, tag_tab, out_ref):
    mesh = plsc.VectorSubcoreMesh(core_axis_name="c", subcore_axis_name="s")
    pl.kernel(
        _sc_scatter_body,
        out_type=(),
        mesh=mesh,
        scratch_types=(
            [pltpu.VMEM((NCH, CH), jnp.int32)]
            + [pltpu.VMEM((CH, EMB_D), jnp.float32)] * NBUF
            + [pltpu.SemaphoreType.DMA] * (2 * NBUF)
        ),
    )(tag_idx, tag_tab, out_ref)


def _tc_body(x_ref, w_ref, b_ref, mask_ref, ptab_ref, out_ref):
    acc = jnp.dot(x_ref[...], w_ref[...], preferred_element_type=jnp.float32)
    out_ref[:, :PROJ_D] = acc + b_ref[...]
    pred = jnp.where(mask_ref[...] == 0, ptab_ref[0:1, :], ptab_ref[1:2, :])
    out_ref[:, PROJ_D + EMB_D:] = pred


def _tc_project(x2d, W, b2d, mask_col, ptab):
    return pl.pallas_call(
        _tc_body,
        grid=(TOK // M_TILE,),
        in_specs=[
            pl.BlockSpec((M_TILE, IN_D), lambda i: (i, 0)),
            pl.BlockSpec((IN_D, PROJ_D), lambda i: (0, 0)),
            pl.BlockSpec((1, PROJ_D), lambda i: (0, 0)),
            pl.BlockSpec((M_TILE, 1), lambda i: (i, 0)),
            pl.BlockSpec((2, EMB_D), lambda i: (0, 0)),
        ],
        out_specs=pl.BlockSpec((M_TILE, OUT_D), lambda i: (i, 0)),
        out_shape=jax.ShapeDtypeStruct((TOK, OUT_D), jnp.float32),
    )(x2d, W, b2d, mask_col, ptab)


def kernel(input_layer, tag_ids, predicate_mask, tag_embeddings,
           predicate_embeddings, W, b):
    x2d = input_layer.reshape(TOK, IN_D)
    tag_idx = tag_ids.astype(jnp.int32).reshape(NW * NCH, CH)
    mask_col = predicate_mask.astype(jnp.int32).reshape(TOK, 1)
    out = _tc_project(x2d, W, b.reshape(1, PROJ_D), mask_col,
                      predicate_embeddings)
    out_ref = jax.new_ref(out)
    _sc_scatter_tags(tag_idx, tag_embeddings, out_ref)
    return out_ref[...].reshape(B, S, OUT_D)
